# Initial kernel scaffold; baseline (speedup 1.0000x reference)
import jax
import jax.numpy as jnp
from jax.experimental import pallas as pl


def kernel(text_features, audio_features, video_features, type_emb, W0, att_src0, att_dst0, b0, W1, att_src1, att_dst1, b1, Wout, bout):
    raise NotImplementedError("write your pallas kernel here")


# speedup vs baseline: 148.2568x; 148.2568x over previous
import jax
import jax.numpy as jnp
from jax.experimental import pallas as pl
from jax.experimental.pallas import tpu as pltpu

B = 8192
D = 128
G = 128
H = 4
NEG = 0.2

BB = 1024


def _gat_layer(xs, W_ref, asrc_ref, adst_ref, b_ref):
    W = W_ref[:]
    hs = [jnp.dot(x, W, preferred_element_type=jnp.float32) for x in xs]
    hc = [[h[:, k * G:(k + 1) * G] for k in range(H)] for h in hs]
    al_s = [[jnp.sum(hc[i][k] * asrc_ref[k:k + 1, :], axis=1, keepdims=True)
             for k in range(H)] for i in range(3)]
    al_d = [[jnp.sum(hc[i][k] * adst_ref[k:k + 1, :], axis=1, keepdims=True)
             for k in range(H)] for i in range(3)]
    outs = []
    for j in range(3):
        acc = None
        for k in range(H):
            e = [al_s[i][k] + al_d[j][k] for i in range(3)]
            e = [jnp.where(v > 0, v, NEG * v) for v in e]
            m = jnp.maximum(jnp.maximum(e[0], e[1]), e[2])
            ex = [jnp.exp(v - m) for v in e]
            den = ex[0] + ex[1] + ex[2] + 1e-16
            chunk = (ex[0] * hc[0][k] + ex[1] * hc[1][k] + ex[2] * hc[2][k]) / den
            acc = chunk if acc is None else acc + chunk
        outs.append(acc * (1.0 / H) + b_ref[0:1, :])
    return outs


def _fused_kernel(t_ref, a_ref, v_ref, te_ref,
                  W0_ref, as0_ref, ad0_ref, b0_ref,
                  W1_ref, as1_ref, ad1_ref, b1_ref,
                  Wout_ref, bout_ref, out_ref):
    xs = [t_ref[:] + te_ref[0:1, :],
          a_ref[:] + te_ref[1:2, :],
          v_ref[:] + te_ref[2:3, :]]
    ys = _gat_layer(xs, W0_ref, as0_ref, ad0_ref, b0_ref)
    ys = [jnp.maximum(y, 0.0) for y in ys]
    zs = _gat_layer(ys, W1_ref, as1_ref, ad1_ref, b1_ref)
    zs = [jnp.maximum(z, 0.0) for z in zs]
    pooled = (zs[0] + zs[1] + zs[2]) * (1.0 / 3.0)
    out_ref[:] = jnp.dot(pooled, Wout_ref[:],
                         preferred_element_type=jnp.float32) + bout_ref[0:1, :]


def kernel(text_features, audio_features, video_features, type_emb,
           W0, att_src0, att_dst0, b0, W1, att_src1, att_dst1, b1, Wout, bout):
    b0_2d = b0.reshape(1, G)
    b1_2d = b1.reshape(1, G)
    bout_2d = bout.reshape(1, D)

    grid = (B // BB,)
    feat_spec = pl.BlockSpec((BB, D), lambda i: (i, 0))
    full = lambda shape: pl.BlockSpec(shape, lambda i: (0,) * len(shape))

    return pl.pallas_call(
        _fused_kernel,
        grid=grid,
        in_specs=[
            feat_spec, feat_spec, feat_spec,
            full((3, D)),
            full((D, H * G)), full((H, G)), full((H, G)), full((1, G)),
            full((G, H * G)), full((H, G)), full((H, G)), full((1, G)),
            full((G, D)), full((1, D)),
        ],
        out_specs=pl.BlockSpec((BB, D), lambda i: (i, 0)),
        out_shape=jax.ShapeDtypeStruct((B, D), jnp.float32),
        compiler_params=pltpu.CompilerParams(
            dimension_semantics=("parallel",)),
    )(text_features, audio_features, video_features, type_emb,
      W0, att_src0, att_dst0, b0_2d,
      W1, att_src1, att_dst1, b1_2d,
      Wout, bout_2d)


# speedup vs baseline: 187.9667x; 1.2678x over previous
import jax
import jax.numpy as jnp
from jax.experimental import pallas as pl
from jax.experimental.pallas import tpu as pltpu

B = 8192
D = 128
G = 128
H = 4
NEG = 0.2

BB = 1024
HG = H * G
EXT = HG + 2 * G


def _att_mats(att_src, att_dst):
    c = jnp.arange(3 * H)
    mask_src = (c[None, :] % H == jnp.arange(H)[:, None]).astype(jnp.float32)
    Msrc = (att_src[:, :, None] * mask_src[:, None, :]).reshape(HG, 3 * H)
    Msrc = jnp.pad(Msrc, ((0, 0), (0, G - 3 * H)))
    Mdsts = []
    for i in range(3):
        mask_i = (c[None, :] == (i * H + jnp.arange(H)[:, None])).astype(jnp.float32)
        Mi = (att_dst[:, :, None] * mask_i[:, None, :]).reshape(HG, 3 * H)
        Mdsts.append(jnp.pad(Mi, ((0, 0), (0, G - 3 * H))))
    return Msrc, Mdsts


def _ext_weights(W, att_src, att_dst):
    Msrc, Mdsts = _att_mats(att_src, att_dst)
    WMs = W @ Msrc
    return [jnp.concatenate([W, WMs, W @ Mdsts[i]], axis=1) for i in range(3)]


def _gat_layer(xs, Wrefs, b_ref):
    he = [jnp.dot(xs[i], Wrefs[i][:], preferred_element_type=jnp.float32)
          for i in range(3)]
    h = [he[i][:, 0:HG] for i in range(3)]
    dl = he[0][:, HG + G:] + he[1][:, HG + G:] + he[2][:, HG + G:]
    ex = []
    for i in range(3):
        L = he[i][:, HG:HG + G] + dl
        ex.append(jnp.where(L > 0, L, NEG * L))
    m = jnp.maximum(jnp.maximum(ex[0], ex[1]), ex[2])
    ex = [jnp.exp(v - m) for v in ex]
    inv = 1.0 / (ex[0] + ex[1] + ex[2] + 1e-16)
    al = [v * inv for v in ex]
    outs = []
    for j in range(3):
        acc = None
        for k in range(H):
            cidx = j * H + k
            chunk = (al[0][:, cidx:cidx + 1] * h[0][:, k * G:(k + 1) * G]
                     + al[1][:, cidx:cidx + 1] * h[1][:, k * G:(k + 1) * G]
                     + al[2][:, cidx:cidx + 1] * h[2][:, k * G:(k + 1) * G])
            acc = chunk if acc is None else acc + chunk
        outs.append(acc * (1.0 / H) + b_ref[0:1, :])
    return outs


def _fused_kernel(t_ref, a_ref, v_ref, te_ref,
                  W0a_ref, W0b_ref, W0c_ref, b0_ref,
                  W1a_ref, W1b_ref, W1c_ref, b1_ref,
                  Wout_ref, bout_ref, out_ref):
    xs = [t_ref[:] + te_ref[0:1, :],
          a_ref[:] + te_ref[1:2, :],
          v_ref[:] + te_ref[2:3, :]]
    ys = _gat_layer(xs, [W0a_ref, W0b_ref, W0c_ref], b0_ref)
    ys = [jnp.maximum(y, 0.0) for y in ys]
    zs = _gat_layer(ys, [W1a_ref, W1b_ref, W1c_ref], b1_ref)
    zs = [jnp.maximum(z, 0.0) for z in zs]
    pooled = (zs[0] + zs[1] + zs[2]) * (1.0 / 3.0)
    out_ref[:] = jnp.dot(pooled, Wout_ref[:],
                         preferred_element_type=jnp.float32) + bout_ref[0:1, :]


def kernel(text_features, audio_features, video_features, type_emb,
           W0, att_src0, att_dst0, b0, W1, att_src1, att_dst1, b1, Wout, bout):
    W0e = _ext_weights(W0, att_src0, att_dst0)
    W1e = _ext_weights(W1, att_src1, att_dst1)
    b0_2d = b0.reshape(1, G)
    b1_2d = b1.reshape(1, G)
    bout_2d = bout.reshape(1, D)

    grid = (B // BB,)
    feat_spec = pl.BlockSpec((BB, D), lambda i: (i, 0))
    full = lambda shape: pl.BlockSpec(shape, lambda i: (0,) * len(shape))

    return pl.pallas_call(
        _fused_kernel,
        grid=grid,
        in_specs=[
            feat_spec, feat_spec, feat_spec,
            full((3, D)),
            full((D, EXT)), full((D, EXT)), full((D, EXT)), full((1, G)),
            full((G, EXT)), full((G, EXT)), full((G, EXT)), full((1, G)),
            full((G, D)), full((1, D)),
        ],
        out_specs=pl.BlockSpec((BB, D), lambda i: (i, 0)),
        out_shape=jax.ShapeDtypeStruct((B, D), jnp.float32),
        compiler_params=pltpu.CompilerParams(
            dimension_semantics=("parallel",)),
    )(text_features, audio_features, video_features, type_emb,
      W0e[0], W0e[1], W0e[2], b0_2d,
      W1e[0], W1e[1], W1e[2], b1_2d,
      Wout, bout_2d)


# speedup vs baseline: 394.9595x; 2.1012x over previous
import jax
import jax.numpy as jnp
from jax.experimental import pallas as pl
from jax.experimental.pallas import tpu as pltpu

B = 8192
D = 128
G = 128
H = 4
NEG = 0.2

BB = 1024
HG = H * G
NL = 16
EXT = HG + 2 * NL


def _att_mats(att_src, att_dst):
    c = jnp.arange(3 * H)
    mask_src = (c[None, :] % H == jnp.arange(H)[:, None]).astype(jnp.float32)
    Msrc = (att_src[:, :, None] * mask_src[:, None, :]).reshape(HG, 3 * H)
    Msrc = jnp.pad(Msrc, ((0, 0), (0, NL - 3 * H)))
    Mdsts = []
    for i in range(3):
        mask_i = (c[None, :] == (i * H + jnp.arange(H)[:, None])).astype(jnp.float32)
        Mi = (att_dst[:, :, None] * mask_i[:, None, :]).reshape(HG, 3 * H)
        Mdsts.append(jnp.pad(Mi, ((0, 0), (0, NL - 3 * H))))
    return Msrc, Mdsts


def _ext_weights(W, att_src, att_dst):
    Msrc, Mdsts = _att_mats(att_src, att_dst)
    WMs = W @ Msrc
    return [jnp.concatenate([W, WMs, W @ Mdsts[i]], axis=1) for i in range(3)]


def _dotT(A, X, dA, dX):
    return jax.lax.dot_general(A, X, (((dA,), (dX,)), ((), ())),
                               preferred_element_type=jnp.float32)


def _gat_layer(xTs, xdim, Wrefs, b_ref):
    he = [_dotT(Wrefs[i][:], xTs[i], 0, xdim) for i in range(3)]
    h = [he[i][0:HG, :] for i in range(3)]
    dl = (he[0][HG + NL:, :] + he[1][HG + NL:, :]
          + he[2][HG + NL:, :])
    ex = []
    for i in range(3):
        L = he[i][HG:HG + NL, :] + dl
        ex.append(jnp.where(L > 0, L, NEG * L))
    m = jnp.maximum(jnp.maximum(ex[0], ex[1]), ex[2])
    ex = [jnp.exp(v - m) for v in ex]
    inv = 1.0 / (ex[0] + ex[1] + ex[2] + 1e-16)
    al = [v * inv for v in ex]
    accs = [None, None, None]
    for k in range(H):
        hcs = [h[i][k * G:(k + 1) * G, :] for i in range(3)]
        for j in range(3):
            c = j * H + k
            contrib = (al[0][c:c + 1, :] * hcs[0]
                       + al[1][c:c + 1, :] * hcs[1]
                       + al[2][c:c + 1, :] * hcs[2])
            accs[j] = contrib if accs[j] is None else accs[j] + contrib
    return [a * (1.0 / H) + b_ref[:] for a in accs]


def _fused_kernel(t_ref, a_ref, v_ref,
                  W0a_ref, W0b_ref, W0c_ref, b0_ref,
                  W1a_ref, W1b_ref, W1c_ref, b1_ref,
                  Wout_ref, bout_ref, out_ref):
    xs = [t_ref[:], a_ref[:], v_ref[:]]
    ys = _gat_layer(xs, 1, [W0a_ref, W0b_ref, W0c_ref], b0_ref)
    ys = [jnp.maximum(y, 0.0) for y in ys]
    zs = _gat_layer(ys, 0, [W1a_ref, W1b_ref, W1c_ref], b1_ref)
    zs = [jnp.maximum(z, 0.0) for z in zs]
    pooled = (zs[0] + zs[1] + zs[2]) * (1.0 / 3.0)
    out_ref[:] = _dotT(pooled, Wout_ref[:], 0, 0) + bout_ref[:]


def kernel(text_features, audio_features, video_features, type_emb,
           W0, att_src0, att_dst0, b0, W1, att_src1, att_dst1, b1, Wout, bout):
    W0e = _ext_weights(W0, att_src0, att_dst0)
    W1e = _ext_weights(W1, att_src1, att_dst1)
    b0_col = jnp.broadcast_to(b0.reshape(G, 1), (G, 128))
    b1_col = jnp.broadcast_to(b1.reshape(G, 1), (G, 128))
    bout_row = bout.reshape(1, D)

    te_bias = [jnp.broadcast_to((type_emb[i:i + 1, :] @ W0e[i]).reshape(EXT, 1),
                                (EXT, 128)) for i in range(3)]

    grid = (B // BB,)
    feat_spec = pl.BlockSpec((BB, D), lambda i: (i, 0))
    full = lambda shape: pl.BlockSpec(shape, lambda i: (0,) * len(shape))

    def body(t_ref, a_ref, v_ref,
             W0a, W0b, W0c, te0, te1, te2, b0r,
             W1a, W1b, W1c, b1r, Woutr, boutr, out_ref):
        xs = [t_ref[:], a_ref[:], v_ref[:]]
        Wr = [W0a, W0b, W0c]
        ter = [te0, te1, te2]
        he = [_dotT(Wr[i][:], xs[i], 0, 1) + ter[i][:, 0:1] for i in range(3)]
        h = [he[i][0:HG, :] for i in range(3)]
        dl = he[0][HG + NL:, :] + he[1][HG + NL:, :] + he[2][HG + NL:, :]
        ex = []
        for i in range(3):
            L = he[i][HG:HG + NL, :] + dl
            ex.append(jnp.where(L > 0, L, NEG * L))
        m = jnp.maximum(jnp.maximum(ex[0], ex[1]), ex[2])
        ex = [jnp.exp(v - m) for v in ex]
        inv = 1.0 / (ex[0] + ex[1] + ex[2] + 1e-16)
        al = [v * inv for v in ex]
        accs = [None, None, None]
        for k in range(H):
            hcs = [h[i][k * G:(k + 1) * G, :] for i in range(3)]
            for j in range(3):
                c = j * H + k
                contrib = (al[0][c:c + 1, :] * hcs[0]
                           + al[1][c:c + 1, :] * hcs[1]
                           + al[2][c:c + 1, :] * hcs[2])
                accs[j] = contrib if accs[j] is None else accs[j] + contrib
        ys = [jnp.maximum(a * (1.0 / H) + b0r[:, 0:1], 0.0) for a in accs]
        zs = _gat_layer(ys, 0, [W1a, W1b, W1c], b1r[:, 0:1])
        zs = [jnp.maximum(z, 0.0) for z in zs]
        pooled = (zs[0] + zs[1] + zs[2]) * (1.0 / 3.0)
        out_ref[:] = _dotT(pooled, Woutr[:], 0, 0) + boutr[:]

    return pl.pallas_call(
        body,
        grid=grid,
        in_specs=[
            feat_spec, feat_spec, feat_spec,
            full((D, EXT)), full((D, EXT)), full((D, EXT)),
            full((EXT, 128)), full((EXT, 128)), full((EXT, 128)), full((G, 128)),
            full((G, EXT)), full((G, EXT)), full((G, EXT)), full((G, 128)),
            full((G, D)), full((1, D)),
        ],
        out_specs=pl.BlockSpec((BB, D), lambda i: (i, 0)),
        out_shape=jax.ShapeDtypeStruct((B, D), jnp.float32),
        compiler_params=pltpu.CompilerParams(
            dimension_semantics=("parallel",)),
    )(text_features, audio_features, video_features,
      W0e[0], W0e[1], W0e[2], te_bias[0], te_bias[1], te_bias[2], b0_col,
      W1e[0], W1e[1], W1e[2], b1_col,
      Wout, bout_row)
